# Initial kernel scaffold; baseline (speedup 1.0000x reference)
import jax
import jax.numpy as jnp
from jax.experimental import pallas as pl


def kernel(hidden_states, rms_weight, router_weight, w_gate, w_up, w_down):
    raise NotImplementedError("write your pallas kernel here")


# speedup vs baseline: 1.2782x; 1.2782x over previous
import jax
import jax.numpy as jnp
from jax.experimental import pallas as pl
from jax.experimental.pallas import tpu as pltpu

HIDDEN = 1024
FF = 2048
N_EXPERTS = 8
TOP_K = 2
EPS = 1e-6


def _router_body(t_ref, rmsw_ref, rw_ref, h_ref, comb_ref):
    t = t_ref[...]
    var = jnp.mean(t * t, axis=-1, keepdims=True)
    h = t * jax.lax.rsqrt(var + EPS) * rmsw_ref[...]
    h_ref[...] = h
    logits = jax.lax.dot_general(
        h, rw_ref[...], (((1,), (1,)), ((), ())),
        preferred_element_type=jnp.float32)
    e = logits.shape[-1]
    ids = jax.lax.broadcasted_iota(jnp.int32, logits.shape, 1)
    l1 = jnp.max(logits, axis=-1, keepdims=True)
    first = jnp.min(jnp.where(logits == l1, ids, e), axis=-1, keepdims=True)
    masked = jnp.where(ids == first, -jnp.inf, logits)
    l2 = jnp.max(masked, axis=-1, keepdims=True)
    second = jnp.min(jnp.where(masked == l2, ids, e), axis=-1, keepdims=True)
    p2 = jnp.exp(l2 - l1)
    s = 1.0 + p2
    w1 = 1.0 / s
    w2 = p2 / s
    comb_ref[...] = (jnp.where(ids == first, w1, 0.0)
                     + jnp.where(ids == second, w2, 0.0))


def _expert_body(h_ref, comb_ref, wg_ref, wu_ref, wd_ref, out_ref):
    e = pl.program_id(0)
    h = h_ref[...]
    g = jnp.dot(h, wg_ref[0], preferred_element_type=jnp.float32)
    u = jnp.dot(h, wu_ref[0], preferred_element_type=jnp.float32)
    act = (g * jax.lax.logistic(g)) * u
    y = jnp.dot(act, wd_ref[0], preferred_element_type=jnp.float32)
    comb = comb_ref[...]
    ids = jax.lax.broadcasted_iota(jnp.int32, comb.shape, 1)
    w = jnp.sum(jnp.where(ids == e, comb, 0.0), axis=1, keepdims=True)
    contrib = y * w

    @pl.when(e == 0)
    def _init():
        out_ref[...] = contrib

    @pl.when(e != 0)
    def _acc():
        out_ref[...] += contrib


def kernel(hidden_states, rms_weight, router_weight, w_gate, w_up, w_down):
    b, s, d = hidden_states.shape
    t = hidden_states.astype(jnp.bfloat16).astype(jnp.float32).reshape(-1, d)
    n_tok = t.shape[0]

    h, comb = pl.pallas_call(
        _router_body,
        out_shape=[
            jax.ShapeDtypeStruct((n_tok, d), jnp.float32),
            jax.ShapeDtypeStruct((n_tok, N_EXPERTS), jnp.float32),
        ],
    )(t, rms_weight.reshape(1, d), router_weight)

    out = pl.pallas_call(
        _expert_body,
        grid=(N_EXPERTS,),
        in_specs=[
            pl.BlockSpec((n_tok, d), lambda e: (0, 0)),
            pl.BlockSpec((n_tok, N_EXPERTS), lambda e: (0, 0)),
            pl.BlockSpec((1, d, FF), lambda e: (e, 0, 0)),
            pl.BlockSpec((1, d, FF), lambda e: (e, 0, 0)),
            pl.BlockSpec((1, FF, d), lambda e: (e, 0, 0)),
        ],
        out_specs=pl.BlockSpec((n_tok, d), lambda e: (0, 0)),
        out_shape=jax.ShapeDtypeStruct((n_tok, d), jnp.float32),
    )(h, comb, w_gate, w_up, w_down)

    return out.reshape(b, s, d)
